# baseline (speedup 1.0000x reference)
import functools

import jax
import jax.numpy as jnp
from jax import lax
from jax.experimental import pallas as pl
from jax.experimental.pallas import tpu as pltpu
from jax.experimental.pallas import tpu_sc as plsc

_VOCAB = 100000
_HIDDEN = 128
_BATCH = 1024
_SEQ = 50

_NC = 2
_NS = 16
_NW = _NC * _NS
_B_PER_W = _BATCH // _NW
_LANES = 16


def _tree_sum(vals):
    while len(vals) > 1:
        nxt = [vals[i] + vals[i + 1] for i in range(0, len(vals) - 1, 2)]
        if len(vals) % 2:
            nxt.append(vals[-1])
        vals = nxt
    return vals[0]


def _gather_mean(x, table):
    mesh = plsc.VectorSubcoreMesh(core_axis_name="c", subcore_axis_name="s")

    @functools.partial(
        pl.kernel,
        out_type=jax.ShapeDtypeStruct((_BATCH, _HIDDEN), jnp.float32),
        mesh=mesh,
        scratch_types=[
            pltpu.VMEM((_B_PER_W, _SEQ), jnp.int32),
            pltpu.VMEM((_SEQ, _HIDDEN), jnp.float32),
            pltpu.VMEM((_SEQ, _HIDDEN), jnp.float32),
            pltpu.VMEM((_B_PER_W, _HIDDEN), jnp.float32),
            pltpu.SemaphoreType.DMA,
            pltpu.SemaphoreType.DMA,
        ],
    )
    def gm(x_hbm, table_hbm, out_hbm, idx_v, rows0_v, rows1_v, acc_v, sem0, sem1):
        wid = lax.axis_index("s") * _NC + lax.axis_index("c")
        base = wid * _B_PER_W
        pltpu.sync_copy(x_hbm.at[pl.ds(base, _B_PER_W)], idx_v)

        bufs = (rows0_v, rows1_v)
        sems = (sem0, sem1)

        def start(i, slot):
            pltpu.async_copy(table_hbm.at[idx_v.at[i]], bufs[slot], sems[slot])

        def finish(i, slot):
            pltpu.make_async_copy(table_hbm.at[idx_v.at[i]], bufs[slot],
                                  sems[slot]).wait()
            rows = bufs[slot]
            for d in range(_HIDDEN // _LANES):
                sl = pl.ds(d * _LANES, _LANES)
                total = _tree_sum([rows[l, sl] for l in range(_SEQ)])
                acc_v[i, sl] = total * (1.0 / _SEQ)

        start(0, 0)

        def body(k, _):
            i = k * 2
            start(i + 1, 1)
            finish(i, 0)

            @pl.when(i + 2 < _B_PER_W)
            def _():
                start(i + 2, 0)

            finish(i + 1, 1)
            return 0

        lax.fori_loop(0, _B_PER_W // 2, body, 0)
        pltpu.sync_copy(acc_v, out_hbm.at[pl.ds(base, _B_PER_W)])

    return gm(x, table)


_V_BLK = 512


def _mm_body(vec_ref, w_ref, b_ref, out_ref):
    acc = lax.dot_general(
        vec_ref[...].astype(jnp.bfloat16),
        w_ref[...].astype(jnp.bfloat16),
        dimension_numbers=(((1,), (1,)), ((), ())),
        preferred_element_type=jnp.float32,
    )
    out_ref[...] = acc + b_ref[...]


def _linear(vec, W, b):
    grid = (pl.cdiv(_VOCAB, _V_BLK),)
    return pl.pallas_call(
        _mm_body,
        grid=grid,
        in_specs=[
            pl.BlockSpec((_BATCH, _HIDDEN), lambda j: (0, 0)),
            pl.BlockSpec((_V_BLK, _HIDDEN), lambda j: (j, 0)),
            pl.BlockSpec((1, _V_BLK), lambda j: (0, j)),
        ],
        out_specs=pl.BlockSpec((_BATCH, _V_BLK), lambda j: (0, j)),
        out_shape=jax.ShapeDtypeStruct((_BATCH, _VOCAB), jnp.float32),
    )(vec, W, b.reshape(1, _VOCAB))


def kernel(x, table, W, b):
    vec = _gather_mean(x, table)
    return _linear(vec, W, b)


# speedup vs baseline: 1.1551x; 1.1551x over previous
import functools

import jax
import jax.numpy as jnp
from jax import lax
from jax.experimental import pallas as pl
from jax.experimental.pallas import tpu as pltpu
from jax.experimental.pallas import tpu_sc as plsc

_VOCAB = 100000
_HIDDEN = 128
_BATCH = 1024
_SEQ = 50

_NC = 2
_NS = 16
_NW = _NC * _NS
_B_PER_W = _BATCH // _NW
_LANES = 16


def _tree_sum(vals):
    while len(vals) > 1:
        nxt = [vals[i] + vals[i + 1] for i in range(0, len(vals) - 1, 2)]
        if len(vals) % 2:
            nxt.append(vals[-1])
        vals = nxt
    return vals[0]


def _gather_mean(x, table):
    mesh = plsc.VectorSubcoreMesh(core_axis_name="c", subcore_axis_name="s")

    @functools.partial(
        pl.kernel,
        out_type=jax.ShapeDtypeStruct((_BATCH, _HIDDEN), jnp.float32),
        mesh=mesh,
        scratch_types=[
            pltpu.VMEM((_B_PER_W, _SEQ), jnp.int32),
            pltpu.VMEM((_SEQ, _HIDDEN), jnp.float32),
            pltpu.VMEM((_SEQ, _HIDDEN), jnp.float32),
            pltpu.VMEM((_B_PER_W, _HIDDEN), jnp.float32),
            pltpu.SemaphoreType.DMA,
            pltpu.SemaphoreType.DMA,
        ],
    )
    def gm(x_hbm, table_hbm, out_hbm, idx_v, rows0_v, rows1_v, acc_v, sem0, sem1):
        wid = lax.axis_index("s") * _NC + lax.axis_index("c")
        base = wid * _B_PER_W
        pltpu.sync_copy(x_hbm.at[pl.ds(base, _B_PER_W)], idx_v)

        bufs = (rows0_v, rows1_v)
        sems = (sem0, sem1)

        def start(i, slot):
            pltpu.async_copy(table_hbm.at[idx_v.at[i]], bufs[slot], sems[slot])

        def finish(i, slot):
            pltpu.make_async_copy(table_hbm.at[idx_v.at[i]], bufs[slot],
                                  sems[slot]).wait()
            rows = bufs[slot]
            for d in range(_HIDDEN // _LANES):
                sl = pl.ds(d * _LANES, _LANES)
                total = _tree_sum([rows[l, sl] for l in range(_SEQ)])
                acc_v[i, sl] = total * (1.0 / _SEQ)

        start(0, 0)

        def body(k, _):
            i = k * 2
            start(i + 1, 1)
            finish(i, 0)

            @pl.when(i + 2 < _B_PER_W)
            def _():
                start(i + 2, 0)

            finish(i + 1, 1)
            return 0

        lax.fori_loop(0, _B_PER_W // 2, body, 0)
        pltpu.sync_copy(acc_v, out_hbm.at[pl.ds(base, _B_PER_W)])

    return gm(x, table)


_V_BLK = 2048


def _mm_body(vec_ref, w_ref, b_ref, out_ref):
    acc = lax.dot_general(
        vec_ref[...].astype(jnp.bfloat16),
        w_ref[...].astype(jnp.bfloat16),
        dimension_numbers=(((1,), (1,)), ((), ())),
        preferred_element_type=jnp.float32,
    )
    out_ref[...] = acc + b_ref[...]


def _linear(vec, W, b):
    grid = (pl.cdiv(_VOCAB, _V_BLK),)
    return pl.pallas_call(
        _mm_body,
        grid=grid,
        in_specs=[
            pl.BlockSpec((_BATCH, _HIDDEN), lambda j: (0, 0)),
            pl.BlockSpec((_V_BLK, _HIDDEN), lambda j: (j, 0)),
            pl.BlockSpec((1, _V_BLK), lambda j: (0, j)),
        ],
        out_specs=pl.BlockSpec((_BATCH, _V_BLK), lambda j: (0, j)),
        out_shape=jax.ShapeDtypeStruct((_BATCH, _VOCAB), jnp.float32),
    )(vec, W, b.reshape(1, _VOCAB))


def kernel(x, table, W, b):
    vec = _gather_mean(x, table)
    return _linear(vec, W, b)
